# baseline (speedup 1.0000x reference)
import jax
import jax.numpy as jnp
from jax.experimental import pallas as pl
from jax.experimental.pallas import tpu as pltpu


def _netvlad_kernel(xa_ref, xb_ref, w_ref, c_ref, o_ref):
    K, C = w_ref.shape
    w_bf = w_ref[...].astype(jnp.bfloat16)
    c = c_ref[...]
    for h, x_ref in enumerate((xa_ref, xb_ref)):
        x_bf = x_ref[0, 0].astype(jnp.bfloat16)
        logits = jnp.dot(w_bf, x_bf, preferred_element_type=jnp.float32)
        m = jnp.max(logits, axis=0, keepdims=True)
        e = jnp.exp(logits - m)
        s = jnp.sum(e, axis=0, keepdims=True)
        a = e / s
        a_bf = a.astype(jnp.bfloat16)
        ones_n = jnp.ones((a_bf.shape[1], 128), dtype=jnp.bfloat16)
        a_sum = jax.lax.dot_general(
            a_bf, ones_n, (((1,), (0,)), ((), ())),
            preferred_element_type=jnp.float32)[:, 0:1]
        ax = jax.lax.dot_general(
            a_bf, x_bf, (((1,), (1,)), ((), ())),
            preferred_element_type=jnp.float32)
        vlad = ax - a_sum * c
        sq = jnp.sum(vlad * vlad)
        inv = 1.0 / jnp.maximum(jnp.sqrt(sq), 1e-12)
        o_ref[h, 0] = vlad * inv


def kernel(x, conv_w, centroids):
    B, C, N = x.shape
    K = conv_w.shape[0]
    H = B // 2
    x4 = x.reshape(2, H, C, N)
    out = pl.pallas_call(
        _netvlad_kernel,
        grid=(H,),
        in_specs=[
            pl.BlockSpec((1, 1, C, N), lambda b: (0, b, 0, 0)),
            pl.BlockSpec((1, 1, C, N), lambda b: (1, b, 0, 0)),
            pl.BlockSpec((K, C), lambda b: (0, 0)),
            pl.BlockSpec((K, C), lambda b: (0, 0)),
        ],
        out_specs=pl.BlockSpec((2, 1, K, C), lambda b: (0, b, 0, 0)),
        out_shape=jax.ShapeDtypeStruct((2, H, K, C), jnp.float32),
        compiler_params=pltpu.CompilerParams(
            dimension_semantics=("arbitrary",),
        ),
    )(x4, x4, conv_w, centroids)
    return out.reshape(B, K * C)


# speedup vs baseline: 1.0190x; 1.0190x over previous
import jax
import jax.numpy as jnp
from jax.experimental import pallas as pl
from jax.experimental.pallas import tpu as pltpu


def _netvlad_kernel(xa_ref, xb_ref, w_ref, c_ref, o_ref):
    K, C = w_ref.shape
    w_bf = w_ref[...].astype(jnp.bfloat16)
    c = c_ref[...]
    x_bf = [r[0, 0].astype(jnp.bfloat16) for r in (xa_ref, xb_ref)]
    logits = [jnp.dot(w_bf, xb, preferred_element_type=jnp.float32)
              for xb in x_bf]
    a_bf, a_sum = [], []
    for lg in logits:
        m = jnp.max(lg, axis=0, keepdims=True)
        e = jnp.exp(lg - m)
        s = jnp.sum(e, axis=0, keepdims=True)
        a = e / s
        a_sum.append(jnp.sum(a, axis=1, keepdims=True))
        a_bf.append(a.astype(jnp.bfloat16))
    ax = [jax.lax.dot_general(a_bf[h], x_bf[h], (((1,), (1,)), ((), ())),
                              preferred_element_type=jnp.float32)
          for h in range(2)]
    for h in range(2):
        vlad = ax[h] - a_sum[h] * c
        sq = jnp.sum(vlad * vlad)
        inv = 1.0 / jnp.maximum(jnp.sqrt(sq), 1e-12)
        o_ref[h, 0] = vlad * inv


def kernel(x, conv_w, centroids):
    B, C, N = x.shape
    K = conv_w.shape[0]
    H = B // 2
    x4 = x.reshape(2, H, C, N)
    out = pl.pallas_call(
        _netvlad_kernel,
        grid=(H,),
        in_specs=[
            pl.BlockSpec((1, 1, C, N), lambda b: (0, b, 0, 0)),
            pl.BlockSpec((1, 1, C, N), lambda b: (1, b, 0, 0)),
            pl.BlockSpec((K, C), lambda b: (0, 0)),
            pl.BlockSpec((K, C), lambda b: (0, 0)),
        ],
        out_specs=pl.BlockSpec((2, 1, K, C), lambda b: (0, b, 0, 0)),
        out_shape=jax.ShapeDtypeStruct((2, H, K, C), jnp.float32),
        compiler_params=pltpu.CompilerParams(
            dimension_semantics=("arbitrary",),
        ),
    )(x4, x4, conv_w, centroids)
    return out.reshape(B, K * C)
